# baseline (speedup 1.0000x reference)
import functools

import jax
import jax.numpy as jnp
from jax import lax
from jax.experimental import pallas as pl
from jax.experimental.pallas import tpu as pltpu
from jax.experimental.pallas import tpu_sc as plsc

_NC = 2
_NS = 16
_NW = _NC * _NS
_LANES = 128
_SUBL = 8


def _gather_call(n_rows, n_cols, n_table, block_cols):
    n_strips = n_rows // _SUBL
    blocks_per_strip = n_cols // block_cols
    n_blocks = n_strips * blocks_per_strip
    blocks_per_w = n_blocks // _NW
    segs = block_cols // _LANES
    stage = 8000
    n_stage = n_table // stage
    mesh = plsc.VectorSubcoreMesh(core_axis_name="c", subcore_axis_name="s")

    @functools.partial(
        pl.kernel,
        mesh=mesh,
        out_type=jax.ShapeDtypeStruct((n_rows, n_cols), jnp.float32),
        scratch_types=[
            pltpu.VMEM_SHARED((n_table,), jnp.float32),
            pltpu.VMEM((stage,), jnp.float32),
            pltpu.VMEM((stage,), jnp.float32),
            pltpu.VMEM((_SUBL, block_cols), jnp.int32),
            pltpu.VMEM((_SUBL, block_cols), jnp.int32),
            pltpu.VMEM((_SUBL, block_cols), jnp.float32),
            pltpu.VMEM((_SUBL, block_cols), jnp.float32),
            pltpu.SemaphoreType.DMA,
            pltpu.SemaphoreType.DMA,
            pltpu.SemaphoreType.DMA,
            pltpu.SemaphoreType.DMA,
            pltpu.SemaphoreType.DMA,
            pltpu.SemaphoreType.DMA,
            pltpu.SemaphoreType.DMA,
        ],
    )
    def k(v_hbm, idx2d_hbm, out2d_hbm, tab_sp, stg_v0, stg_v1, idx_v0,
          idx_v1, val_v0, val_v1, sem, si0, si1, so0, so1, st0, st1):
        cid = lax.axis_index("c")
        sid = lax.axis_index("s")
        wid = sid * _NC + cid

        start = (n_blocks * wid) // _NW
        end = (n_blocks * (wid + 1)) // _NW
        n_w = end - start
        max_w = -(-n_blocks // _NW)

        def hbm_slice3(ref, q):
            t = q // blocks_per_strip
            b = q % blocks_per_strip
            return ref.at[t, :, pl.ds(b * block_cols, block_cols)]

        idx_hbm3 = idx2d_hbm.reshape(n_strips, _SUBL, n_cols)
        pltpu.async_copy(hbm_slice3(idx_hbm3, start), idx_v0, si0)

        n_rounds = (n_stage + _NS - 1) // _NS
        stgs = (stg_v0, stg_v1)
        sts = (st0, st1)

        @pl.when(sid < n_stage)
        def _():
            pltpu.async_copy(
                v_hbm.at[pl.ds(sid * stage, stage)], stg_v0, st0)

        for j in range(n_rounds):
            p = j % 2
            c = sid + j * _NS
            c1 = sid + (j + 1) * _NS
            if j + 1 < n_rounds:
                @pl.when(c1 < n_stage)
                def _():
                    pltpu.async_copy(
                        v_hbm.at[pl.ds(c1 * stage, stage)],
                        stgs[1 - p], sts[1 - p])

            @pl.when(c < n_stage)
            def _():
                pltpu.make_async_copy(
                    v_hbm.at[pl.ds(c * stage, stage)],
                    stgs[p], sts[p]).wait()
                pltpu.sync_copy(stgs[p], tab_sp.at[pl.ds(c * stage, stage)])

        plsc.subcore_barrier()

        idx_hbm = idx2d_hbm.reshape(n_strips, _SUBL, n_cols)
        out_hbm = out2d_hbm.reshape(n_strips, _SUBL, n_cols)

        def hbm_slice(ref, q):
            t = q // blocks_per_strip
            b = q % blocks_per_strip
            return ref.at[t, :, pl.ds(b * block_cols, block_cols)]

        def process(i, idx_cur, val_cur, idx_nxt, s_in, s_in_nxt, s_out):
            q = start + i

            @pl.when(i + 1 < n_w)
            def _():
                pltpu.async_copy(hbm_slice(idx_hbm, q + 1), idx_nxt,
                                 s_in_nxt)

            pltpu.make_async_copy(hbm_slice(idx_hbm, q), idx_cur,
                                  s_in).wait()

            @pl.when(i >= 2)
            def _():
                pltpu.make_async_copy(val_cur, hbm_slice(out_hbm, q),
                                      s_out).wait()

            descs = []
            for r in range(_SUBL):
                for s in range(segs):
                    descs.append(pltpu.async_copy(
                        tab_sp.at[idx_cur.at[r, pl.ds(s * _LANES, _LANES)]],
                        val_cur.at[r, pl.ds(s * _LANES, _LANES)],
                        sem,
                    ))
            for d in descs:
                d.wait()

            pltpu.async_copy(val_cur, hbm_slice(out_hbm, q), s_out)

        def body(i, carry):
            @pl.when((lax.rem(i, 2) == 0) & (i < n_w))
            def _():
                process(i, idx_v0, val_v0, idx_v1, si0, si1, so0)

            @pl.when((lax.rem(i, 2) == 1) & (i < n_w))
            def _():
                process(i, idx_v1, val_v1, idx_v0, si1, si0, so1)

            return carry

        lax.fori_loop(0, max_w, body, 0)

        q_last = end - 1

        @pl.when(lax.rem(n_w, 2) == 1)
        def _():
            pltpu.make_async_copy(val_v1, hbm_slice(out_hbm, q_last - 1),
                                  so1).wait()
            pltpu.make_async_copy(val_v0, hbm_slice(out_hbm, q_last),
                                  so0).wait()

        @pl.when(lax.rem(n_w, 2) == 0)
        def _():
            pltpu.make_async_copy(val_v0, hbm_slice(out_hbm, q_last - 1),
                                  so0).wait()
            pltpu.make_async_copy(val_v1, hbm_slice(out_hbm, q_last),
                                  so1).wait()

    return k


def kernel(v, idx):
    b, s = idx.shape
    out_t = _gather_call(s, b, v.shape[0], 1024)(v, idx.T.astype(jnp.int32))
    return out_t.T


# speedup vs baseline: 1.0023x; 1.0023x over previous
import functools

import jax
import jax.numpy as jnp
from jax import lax
from jax.experimental import pallas as pl
from jax.experimental.pallas import tpu as pltpu
from jax.experimental.pallas import tpu_sc as plsc

_NC = 2
_NS = 16
_NW = _NC * _NS
_LANES = 128
_SUBL = 8


def _gather_call(n_rows, n_cols, n_table, block_cols):
    n_strips = n_rows // _SUBL
    blocks_per_strip = n_cols // block_cols
    n_blocks = n_strips * blocks_per_strip
    blocks_per_w = n_blocks // _NW
    segs = block_cols // _LANES
    stage = 8000
    n_stage = n_table // stage
    mesh = plsc.VectorSubcoreMesh(core_axis_name="c", subcore_axis_name="s")

    @functools.partial(
        pl.kernel,
        mesh=mesh,
        out_type=jax.ShapeDtypeStruct((n_rows, n_cols), jnp.float32),
        scratch_types=[
            pltpu.VMEM_SHARED((n_table,), jnp.float32),
            pltpu.VMEM((stage,), jnp.float32),
            pltpu.VMEM((stage,), jnp.float32),
            pltpu.VMEM((_SUBL, block_cols), jnp.int32),
            pltpu.VMEM((_SUBL, block_cols), jnp.int32),
            pltpu.VMEM((_SUBL, block_cols), jnp.float32),
            pltpu.VMEM((_SUBL, block_cols), jnp.float32),
            pltpu.SemaphoreType.DMA,
            pltpu.SemaphoreType.DMA,
            pltpu.SemaphoreType.DMA,
            pltpu.SemaphoreType.DMA,
            pltpu.SemaphoreType.DMA,
            pltpu.SemaphoreType.DMA,
            pltpu.SemaphoreType.DMA,
        ],
    )
    def k(v_hbm, idx2d_hbm, out2d_hbm, tab_sp, stg_v0, stg_v1, idx_v0,
          idx_v1, val_v0, val_v1, sem, si0, si1, so0, so1, st0, st1):
        cid = lax.axis_index("c")
        sid = lax.axis_index("s")
        wid = sid * _NC + cid

        blocks_per_w = n_blocks // _NW
        base = wid * blocks_per_w

        def hbm_slice3(ref, q):
            t = q // blocks_per_strip
            b = q % blocks_per_strip
            return ref.at[t, :, pl.ds(b * block_cols, block_cols)]

        idx_hbm3 = idx2d_hbm.reshape(n_strips, _SUBL, n_cols)
        pltpu.async_copy(hbm_slice3(idx_hbm3, base), idx_v0, si0)

        n_rounds = (n_stage + _NS - 1) // _NS
        stgs = (stg_v0, stg_v1)
        sts = (st0, st1)

        @pl.when(sid < n_stage)
        def _():
            pltpu.async_copy(
                v_hbm.at[pl.ds(sid * stage, stage)], stg_v0, st0)

        for j in range(n_rounds):
            p = j % 2
            c = sid + j * _NS
            c1 = sid + (j + 1) * _NS
            if j + 1 < n_rounds:
                @pl.when(c1 < n_stage)
                def _():
                    pltpu.async_copy(
                        v_hbm.at[pl.ds(c1 * stage, stage)],
                        stgs[1 - p], sts[1 - p])

            @pl.when(c < n_stage)
            def _():
                pltpu.make_async_copy(
                    v_hbm.at[pl.ds(c * stage, stage)],
                    stgs[p], sts[p]).wait()
                pltpu.sync_copy(stgs[p], tab_sp.at[pl.ds(c * stage, stage)])

        plsc.subcore_barrier()

        idx_hbm = idx2d_hbm.reshape(n_strips, _SUBL, n_cols)
        out_hbm = out2d_hbm.reshape(n_strips, _SUBL, n_cols)

        base = wid * blocks_per_w

        def hbm_slice(ref, q):
            t = q // blocks_per_strip
            b = q % blocks_per_strip
            return ref.at[t, :, pl.ds(b * block_cols, block_cols)]

        def process(i, idx_cur, val_cur, idx_nxt, s_in, s_in_nxt, s_out):
            q = base + i

            @pl.when(i + 1 < blocks_per_w)
            def _():
                pltpu.async_copy(hbm_slice(idx_hbm, q + 1), idx_nxt,
                                 s_in_nxt)

            pltpu.make_async_copy(hbm_slice(idx_hbm, q), idx_cur,
                                  s_in).wait()

            @pl.when(i >= 2)
            def _():
                pltpu.make_async_copy(val_cur, hbm_slice(out_hbm, q),
                                      s_out).wait()

            descs = []
            for r in range(_SUBL):
                for s in range(segs):
                    descs.append(pltpu.async_copy(
                        tab_sp.at[idx_cur.at[r, pl.ds(s * _LANES, _LANES)]],
                        val_cur.at[r, pl.ds(s * _LANES, _LANES)],
                        sem,
                    ))
            for d in descs:
                d.wait()

            pltpu.async_copy(val_cur, hbm_slice(out_hbm, q), s_out)

        def body(i, carry):
            @pl.when(lax.rem(i, 2) == 0)
            def _():
                process(i, idx_v0, val_v0, idx_v1, si0, si1, so0)

            @pl.when(lax.rem(i, 2) == 1)
            def _():
                process(i, idx_v1, val_v1, idx_v0, si1, si0, so1)

            return carry

        lax.fori_loop(0, blocks_per_w, body, 0)

        q_last = base + blocks_per_w - 1
        pltpu.make_async_copy(val_v1, hbm_slice(out_hbm, q_last - 1),
                              so1).wait()
        pltpu.make_async_copy(val_v0, hbm_slice(out_hbm, q_last),
                              so0).wait()

    return k


def kernel(v, idx):
    b, s = idx.shape
    out_t = _gather_call(s, b, v.shape[0], 512)(v, idx.T.astype(jnp.int32))
    return out_t.T


# speedup vs baseline: 1.0031x; 1.0008x over previous
import functools

import jax
import jax.numpy as jnp
from jax import lax
from jax.experimental import pallas as pl
from jax.experimental.pallas import tpu as pltpu
from jax.experimental.pallas import tpu_sc as plsc

_NC = 2
_NS = 16
_NW = _NC * _NS
_LANES = 128
_SUBL = 8


def _gather_call(n_rows, n_cols, n_table, block_cols):
    n_strips = n_rows // _SUBL
    blocks_per_strip = n_cols // block_cols
    n_blocks = n_strips * blocks_per_strip
    blocks_per_w = n_blocks // _NW
    segs = block_cols // _LANES
    stage = 8000
    n_stage = n_table // stage
    mesh = plsc.VectorSubcoreMesh(core_axis_name="c", subcore_axis_name="s")

    @functools.partial(
        pl.kernel,
        mesh=mesh,
        out_type=jax.ShapeDtypeStruct((n_rows, n_cols), jnp.float32),
        scratch_types=[
            pltpu.VMEM_SHARED((n_table,), jnp.float32),
            pltpu.VMEM((stage,), jnp.float32),
            pltpu.VMEM((stage,), jnp.float32),
            pltpu.VMEM((_SUBL, block_cols), jnp.int32),
            pltpu.VMEM((_SUBL, block_cols), jnp.int32),
            pltpu.VMEM((_SUBL, block_cols), jnp.float32),
            pltpu.VMEM((_SUBL, block_cols), jnp.float32),
            pltpu.SemaphoreType.DMA,
            pltpu.SemaphoreType.DMA,
            pltpu.SemaphoreType.DMA,
            pltpu.SemaphoreType.DMA,
            pltpu.SemaphoreType.DMA,
            pltpu.SemaphoreType.DMA,
            pltpu.SemaphoreType.DMA,
            pltpu.SemaphoreType.DMA,
            pltpu.SemaphoreType.DMA,
        ],
    )
    def k(v_hbm, idx2d_hbm, out2d_hbm, tab_sp, stg_v0, stg_v1, idx_v0,
          idx_v1, val_v0, val_v1, sem, si0, si1, so0, so1, st0, st1,
          sp0, sp1):
        cid = lax.axis_index("c")
        sid = lax.axis_index("s")
        wid = sid * _NC + cid

        blocks_per_w = n_blocks // _NW
        base = wid * blocks_per_w

        def hbm_slice3(ref, q):
            t = q // blocks_per_strip
            b = q % blocks_per_strip
            return ref.at[t, :, pl.ds(b * block_cols, block_cols)]

        idx_hbm3 = idx2d_hbm.reshape(n_strips, _SUBL, n_cols)
        pltpu.async_copy(hbm_slice3(idx_hbm3, base), idx_v0, si0)

        n_rounds = (n_stage + _NS - 1) // _NS
        stgs = (stg_v0, stg_v1)
        sts = (st0, st1)
        sps = (sp0, sp1)

        @pl.when(sid < n_stage)
        def _():
            pltpu.async_copy(
                v_hbm.at[pl.ds(sid * stage, stage)], stg_v0, st0)

        for j in range(n_rounds):
            p = j % 2
            c = sid + j * _NS
            c1 = sid + (j + 1) * _NS
            if j + 1 < n_rounds:
                @pl.when(c1 < n_stage)
                def _():
                    if j >= 1:
                        c_prev = sid + (j - 1) * _NS
                        pltpu.make_async_copy(
                            stgs[1 - p],
                            tab_sp.at[pl.ds(c_prev * stage, stage)],
                            sps[1 - p]).wait()
                    pltpu.async_copy(
                        v_hbm.at[pl.ds(c1 * stage, stage)],
                        stgs[1 - p], sts[1 - p])

            @pl.when(c < n_stage)
            def _():
                pltpu.make_async_copy(
                    v_hbm.at[pl.ds(c * stage, stage)],
                    stgs[p], sts[p]).wait()
                pltpu.async_copy(
                    stgs[p], tab_sp.at[pl.ds(c * stage, stage)], sps[p])

        def drain_write(j):
            cj = sid + j * _NS
            pltpu.make_async_copy(
                stgs[j % 2], tab_sp.at[pl.ds(cj * stage, stage)],
                sps[j % 2]).wait()

        c_n1 = sid + (n_rounds - 1) * _NS

        @pl.when(c_n1 < n_stage)
        def _():
            drain_write(n_rounds - 2)
            drain_write(n_rounds - 1)

        @pl.when(c_n1 >= n_stage)
        def _():
            drain_write(n_rounds - 3)
            drain_write(n_rounds - 2)

        plsc.subcore_barrier()

        idx_hbm = idx2d_hbm.reshape(n_strips, _SUBL, n_cols)
        out_hbm = out2d_hbm.reshape(n_strips, _SUBL, n_cols)

        base = wid * blocks_per_w

        def hbm_slice(ref, q):
            t = q // blocks_per_strip
            b = q % blocks_per_strip
            return ref.at[t, :, pl.ds(b * block_cols, block_cols)]

        def process(i, idx_cur, val_cur, idx_nxt, s_in, s_in_nxt, s_out):
            q = base + i

            @pl.when(i + 1 < blocks_per_w)
            def _():
                pltpu.async_copy(hbm_slice(idx_hbm, q + 1), idx_nxt,
                                 s_in_nxt)

            pltpu.make_async_copy(hbm_slice(idx_hbm, q), idx_cur,
                                  s_in).wait()

            @pl.when(i >= 2)
            def _():
                pltpu.make_async_copy(val_cur, hbm_slice(out_hbm, q),
                                      s_out).wait()

            descs = []
            for r in range(_SUBL):
                for s in range(segs):
                    descs.append(pltpu.async_copy(
                        tab_sp.at[idx_cur.at[r, pl.ds(s * _LANES, _LANES)]],
                        val_cur.at[r, pl.ds(s * _LANES, _LANES)],
                        sem,
                    ))
            for d in descs:
                d.wait()

            pltpu.async_copy(val_cur, hbm_slice(out_hbm, q), s_out)

        def body(i, carry):
            @pl.when(lax.rem(i, 2) == 0)
            def _():
                process(i, idx_v0, val_v0, idx_v1, si0, si1, so0)

            @pl.when(lax.rem(i, 2) == 1)
            def _():
                process(i, idx_v1, val_v1, idx_v0, si1, si0, so1)

            return carry

        lax.fori_loop(0, blocks_per_w, body, 0)

        q_last = base + blocks_per_w - 1
        pltpu.make_async_copy(val_v1, hbm_slice(out_hbm, q_last - 1),
                              so1).wait()
        pltpu.make_async_copy(val_v0, hbm_slice(out_hbm, q_last),
                              so0).wait()

    return k


def kernel(v, idx):
    b, s = idx.shape
    out_t = _gather_call(s, b, v.shape[0], 512)(v, idx.T.astype(jnp.int32))
    return out_t.T
